# Initial kernel scaffold; baseline (speedup 1.0000x reference)
import jax
import jax.numpy as jnp
from jax.experimental import pallas as pl


def kernel(input_features, adj, weight):
    raise NotImplementedError("write your pallas kernel here")


# speedup vs baseline: 60.6514x; 60.6514x over previous
import functools

import jax
import jax.numpy as jnp
from jax.experimental import pallas as pl
from jax.experimental.pallas import tpu as pltpu


def _support_body(x_ref, w_ref, o_ref, *, precision):
    o_ref[...] = jnp.dot(x_ref[...], w_ref[...],
                         preferred_element_type=jnp.float32,
                         precision=precision)


def _agg_body(a_ref, s_ref, o_ref, *, precision):
    o_ref[...] = jnp.dot(a_ref[...], s_ref[...],
                         preferred_element_type=jnp.float32,
                         precision=precision)


def _pick_block(n: int, target: int) -> int:
    best = 8
    for d in range(8, target + 1, 8):
        if n % d == 0:
            best = d
    return best


def kernel(input_features, adj, weight):
    n, f_in = input_features.shape
    f_out = weight.shape[1]
    precision = jax.lax.Precision.HIGHEST

    x32 = input_features.astype(jnp.float32)
    w32 = weight.astype(jnp.float32)
    a32 = adj.astype(jnp.float32)

    support = pl.pallas_call(
        functools.partial(_support_body, precision=precision),
        out_shape=jax.ShapeDtypeStruct((n, f_out), jnp.float32),
    )(x32, w32)

    bm = _pick_block(n, 400)
    zero = jnp.zeros_like
    out32 = pl.pallas_call(
        functools.partial(_agg_body, precision=precision),
        grid=(n // bm,),
        in_specs=[
            pl.BlockSpec((bm, n), lambda i: (i, zero(i))),
            pl.BlockSpec((n, f_out), lambda i: (zero(i), zero(i))),
        ],
        out_specs=pl.BlockSpec((bm, f_out), lambda i: (i, zero(i))),
        out_shape=jax.ShapeDtypeStruct((n, f_out), jnp.float32),
        compiler_params=pltpu.CompilerParams(
            dimension_semantics=("parallel",),
        ),
    )(a32, support)

    return out32.astype(jnp.float64)


# speedup vs baseline: 131.2954x; 2.1648x over previous
import functools

import jax
import jax.numpy as jnp
from jax.experimental import pallas as pl
from jax.experimental.pallas import tpu as pltpu


def _support_body(x_ref, w_ref, o_ref, *, precision):
    o_ref[...] = jnp.dot(x_ref[...], w_ref[...],
                         preferred_element_type=jnp.float32,
                         precision=precision)


def _agg_body(a_ref, s_ref, o_ref, *, precision):
    o_ref[...] = jnp.dot(a_ref[...], s_ref[...],
                         preferred_element_type=jnp.float32,
                         precision=precision)


def _pick_block(n: int, target: int) -> int:
    best = 8
    for d in range(8, target + 1, 8):
        if n % d == 0:
            best = d
    return best


def kernel(input_features, adj, weight):
    n, f_in = input_features.shape
    f_out = weight.shape[1]
    precision = jax.lax.Precision.DEFAULT

    x32 = input_features.astype(jnp.float32)
    w32 = weight.astype(jnp.float32)
    a32 = adj.astype(jnp.float32)

    support = pl.pallas_call(
        functools.partial(_support_body, precision=precision),
        out_shape=jax.ShapeDtypeStruct((n, f_out), jnp.float32),
    )(x32, w32)

    bm = _pick_block(n, 400)
    zero = jnp.zeros_like
    out32 = pl.pallas_call(
        functools.partial(_agg_body, precision=precision),
        grid=(n // bm,),
        in_specs=[
            pl.BlockSpec((bm, n), lambda i: (i, zero(i))),
            pl.BlockSpec((n, f_out), lambda i: (zero(i), zero(i))),
        ],
        out_specs=pl.BlockSpec((bm, f_out), lambda i: (i, zero(i))),
        out_shape=jax.ShapeDtypeStruct((n, f_out), jnp.float32),
        compiler_params=pltpu.CompilerParams(
            dimension_semantics=("parallel",),
        ),
    )(a32, support)

    return out32.astype(jnp.float64)
